# Initial kernel scaffold; baseline (speedup 1.0000x reference)
import jax
import jax.numpy as jnp
from jax.experimental import pallas as pl


def kernel(inputs, node_feats, A, knn_inds, conv_w, conv_b, w1, b1, w2, b2, w3, b3, w4, b4, wc1, bc1, prelu_a, wc2, bc2):
    raise NotImplementedError("write your pallas kernel here")


# speedup vs baseline: 1.9504x; 1.9504x over previous
import functools

import jax
import jax.numpy as jnp
from jax.experimental import pallas as pl

G, N, K = 2048, 40, 8
C_IN, C_OUT = 32, 6
H = W = 512
D_IN = 576

GB = 64
STATS_ROWS = 8192


def _stats_kernel(x_ref, out_ref):
    i = pl.program_id(0)
    x = x_ref[...]
    s = jnp.sum(x, axis=0, keepdims=True)
    s2 = jnp.sum(x * x, axis=0, keepdims=True)
    part = jnp.concatenate([s, s2], axis=0)

    @pl.when(i == 0)
    def _():
        out_ref[...] = jnp.zeros_like(out_ref)

    out_ref[...] += part


def _layer(xf, a_blk, w_ref, b_ref, d_in, f_out):
    xb = xf.astype(jnp.bfloat16)
    pa = jnp.dot(xb, w_ref[:d_in, :], preferred_element_type=jnp.float32)
    pb = jnp.dot(xb, w_ref[d_in:, :], preferred_element_type=jnp.float32)
    pb3 = pb.reshape(GB, N, f_out)
    agg = jax.lax.dot_general(
        a_blk, pb3.astype(jnp.bfloat16),
        dimension_numbers=(((2,), (1,)), ((0,), (0,))),
        preferred_element_type=jnp.float32)
    h = pa.reshape(GB, N, f_out) + agg + b_ref[...]
    return jnp.maximum(h, 0.0).reshape(GB * N, f_out)


def _gcn_kernel(x_ref, a_ref, knn_ref, stats_ref,
                w1_ref, b1_ref, w2_ref, b2_ref, w3_ref, b3_ref, w4_ref, b4_ref,
                wc1_ref, bc1_ref, pa_ref, wc2_ref, bc2_ref,
                out_ref):
    total = float(G * N)
    mean = stats_ref[0, :] / total
    var = stats_ref[1, :] / total - mean * mean
    rinv = jax.lax.rsqrt(var + 1e-5)

    x = (x_ref[...] - mean) * rinv
    a_blk = a_ref[...].astype(jnp.bfloat16)

    xf = x.reshape(GB * N, D_IN)
    xf = _layer(xf, a_blk, w1_ref, b1_ref, D_IN, 512)
    xf = _layer(xf, a_blk, w2_ref, b2_ref, 512, 256)
    xf = _layer(xf, a_blk, w3_ref, b3_ref, 256, 128)
    xf = _layer(xf, a_blk, w4_ref, b4_ref, 128, 64)

    x4 = xf.reshape(GB, N, 64)
    ids = knn_ref[...]
    iota_n = jax.lax.broadcasted_iota(jnp.int32, (GB, N), 1)
    edges = []
    for k in range(K):
        mask = (iota_n == ids[:, k][:, None]).astype(jnp.float32)
        edges.append(jnp.sum(mask[:, :, None] * x4, axis=1))
    ef = jnp.stack(edges, axis=1).reshape(GB * K, 64)

    h = jnp.dot(ef, wc1_ref[...], preferred_element_type=jnp.float32) + bc1_ref[...]
    h = jnp.where(h >= 0, h, pa_ref[...] * h)
    out_ref[...] = (jnp.dot(h, wc2_ref[...], preferred_element_type=jnp.float32)
                    + bc2_ref[...])


def _conv_kernel(x_ref, w_ref, b_ref, out_ref):
    out_ref[...] = (jnp.dot(w_ref[...], x_ref[...],
                            preferred_element_type=jnp.float32) + b_ref[...])


def kernel(inputs, node_feats, A, knn_inds, conv_w, conv_b,
           w1, b1, w2, b2, w3, b3, w4, b4, wc1, bc1, prelu_a, wc2, bc2):
    flat = node_feats.reshape(G * N, D_IN)
    stats = pl.pallas_call(
        _stats_kernel,
        grid=((G * N) // STATS_ROWS,),
        in_specs=[pl.BlockSpec((STATS_ROWS, D_IN), lambda i: (i, 0))],
        out_specs=pl.BlockSpec((2, D_IN), lambda i: (0, 0)),
        out_shape=jax.ShapeDtypeStruct((2, D_IN), jnp.float32),
    )(flat)

    w1b = w1.astype(jnp.bfloat16)
    w2b = w2.astype(jnp.bfloat16)
    w3b = w3.astype(jnp.bfloat16)
    w4b = w4.astype(jnp.bfloat16)
    const = lambda shape: pl.BlockSpec(shape, lambda i: tuple(0 for _ in shape))
    gcn_pred = pl.pallas_call(
        _gcn_kernel,
        grid=(G // GB,),
        in_specs=[
            pl.BlockSpec((GB, N, D_IN), lambda i: (i, 0, 0)),
            pl.BlockSpec((GB, N, N), lambda i: (i, 0, 0)),
            pl.BlockSpec((GB, K), lambda i: (i, 0)),
            const((2, D_IN)),
            const((2 * D_IN, 512)), const((512,)),
            const((1024, 256)), const((256,)),
            const((512, 128)), const((128,)),
            const((256, 64)), const((64,)),
            const((64, 32)), const((32,)), const((32,)),
            const((32, 2)), const((2,)),
        ],
        out_specs=pl.BlockSpec((GB * K, 2), lambda i: (i, 0)),
        out_shape=jax.ShapeDtypeStruct((G * K, 2), jnp.float32),
    )(node_feats, A, knn_inds, stats,
      w1b, b1, w2b, b2, w3b, b3, w4b, b4, wc1, bc1, prelu_a, wc2, bc2)

    HWB = 16384
    x2 = inputs.reshape(C_IN, H * W)
    pred = pl.pallas_call(
        _conv_kernel,
        grid=((H * W) // HWB,),
        in_specs=[
            pl.BlockSpec((C_IN, HWB), lambda i: (0, i)),
            pl.BlockSpec((C_OUT, C_IN), lambda i: (0, 0)),
            pl.BlockSpec((C_OUT, 1), lambda i: (0, 0)),
        ],
        out_specs=pl.BlockSpec((C_OUT, HWB), lambda i: (0, i)),
        out_shape=jax.ShapeDtypeStruct((C_OUT, H * W), jnp.float32),
    )(x2, conv_w, conv_b.reshape(C_OUT, 1))
    pred_maps = pred.reshape(1, C_OUT, H, W)

    return (pred_maps, gcn_pred)
